# baseline (speedup 1.0000x reference)
import functools

import jax
import jax.numpy as jnp
from jax import lax
from jax.experimental import pallas as pl
from jax.experimental.pallas import tpu as pltpu
from jax.experimental.pallas import tpu_sc as plsc

VOCAB = 1000000
DIM = 32
BATCH = 16384
NEG = 20

NW = 32
B_PER_W = BATCH // NW
CHUNK = 64
NCHUNK = B_PER_W // CHUNK
NEG_ROWS = CHUNK * NEG
NEG_GATHERS = NEG_ROWS // 128


def _sc_scores(input_emb, output_emb, cidx, xidx, nidx):
    mesh = plsc.VectorSubcoreMesh(core_axis_name="c", subcore_axis_name="s")

    @functools.partial(
        pl.kernel,
        mesh=mesh,
        compiler_params=pltpu.CompilerParams(
            needs_layout_passes=False, use_tc_tiling_on_sc=False
        ),
        out_type=jax.ShapeDtypeStruct((NW, 1 + NEG, B_PER_W), jnp.float32),
        scratch_types=[
            pltpu.VMEM((NCHUNK, CHUNK), jnp.int32),
            pltpu.VMEM((NCHUNK, CHUNK), jnp.int32),
            pltpu.VMEM((NCHUNK, NEG_GATHERS, 128), jnp.int32),
            pltpu.VMEM((CHUNK, DIM), jnp.float32),
            pltpu.VMEM((CHUNK, DIM), jnp.float32),
            pltpu.VMEM((NEG_ROWS, DIM), jnp.float32),
            pltpu.VMEM((1 + NEG, B_PER_W), jnp.float32),
            pltpu.SemaphoreType.DMA,
        ],
    )
    def k(in_emb, out_emb, cidx_h, xidx_h, nidx_h, scores_h,
          cidx_v, xidx_v, nidx_v, ctr_v, ctx_v, neg_v, sc_v, sem):
        wid = lax.axis_index("s") * 2 + lax.axis_index("c")
        pltpu.sync_copy(cidx_h.at[wid], cidx_v)
        pltpu.sync_copy(xidx_h.at[wid], xidx_v)
        pltpu.sync_copy(nidx_h.at[wid], nidx_v)

        iota = lax.iota(jnp.int32, 16)

        def chunk_body(c, _):
            cpys = [
                pltpu.async_copy(in_emb.at[cidx_v.at[c]], ctr_v, sem),
                pltpu.async_copy(out_emb.at[xidx_v.at[c]], ctx_v, sem),
            ]
            for j in range(NEG_GATHERS):
                cpys.append(
                    pltpu.async_copy(
                        out_emb.at[nidx_v.at[c, j]],
                        neg_v.at[pl.ds(j * 128, 128)],
                        sem,
                    )
                )
            for cp in cpys:
                cp.wait()

            def group_body(g, _):
                rb = g * 16
                ob = c * CHUNK + rb
                row16 = rb + iota
                nbase = row16 * NEG
                acc_p = jnp.zeros((16,), jnp.float32)
                acc_n = [jnp.zeros((16,), jnp.float32) for _ in range(NEG)]
                for d in range(DIM):
                    colv = jnp.full((16,), d, jnp.int32)
                    cd = plsc.load_gather(ctr_v, [row16, colv])
                    xd = plsc.load_gather(ctx_v, [row16, colv])
                    acc_p = acc_p + cd * xd
                    for kk in range(NEG):
                        nd = plsc.load_gather(neg_v, [nbase + kk, colv])
                        acc_n[kk] = acc_n[kk] - cd * nd
                sc_v[0, pl.ds(ob, 16)] = acc_p
                for kk in range(NEG):
                    sc_v[1 + kk, pl.ds(ob, 16)] = acc_n[kk]
                return ()

            lax.fori_loop(0, CHUNK // 16, group_body, ())
            return ()

        lax.fori_loop(0, NCHUNK, chunk_body, ())
        pltpu.sync_copy(sc_v, scores_h.at[wid])

    return k(input_emb, output_emb, cidx, xidx, nidx)


def _tc_loss(scores):
    def body(s_ref, o_ref):
        x = s_ref[...]
        sig = 1.0 / (1.0 + jnp.exp(-x))
        o_ref[0, 0] = -jnp.sum(jnp.log(sig + 1e-9)) / BATCH

    return pl.pallas_call(
        body,
        out_shape=jax.ShapeDtypeStruct((1, 1), jnp.float32),
        out_specs=pl.BlockSpec(memory_space=pltpu.SMEM),
    )(scores)


def kernel(input_embeddings, output_embeddings, center_words, context_words,
           negative_words):
    cidx = center_words.astype(jnp.int32).reshape(NW, NCHUNK, CHUNK)
    xidx = context_words.astype(jnp.int32).reshape(NW, NCHUNK, CHUNK)
    nidx = negative_words.astype(jnp.int32).reshape(NW, NCHUNK, NEG_GATHERS, 128)
    scores = _sc_scores(input_embeddings, output_embeddings, cidx, xidx, nidx)
    loss = _tc_loss(scores)
    return loss[0, 0]
